# baseline (speedup 1.0000x reference)
import functools

import jax
import jax.numpy as jnp
from jax import lax
from jax.experimental import pallas as pl
from jax.experimental.pallas import tpu as pltpu
from jax.experimental.pallas import tpu_sc as plsc

D = 32
CHUNK = 128
LOOKAHEAD = 4
NBUF = 6
OB = 2


@functools.lru_cache(maxsize=None)
def _build_table_detranspose(vocab: int):
    lanes = 1024
    rows_out = vocab * D // 128
    grid = (vocab + lanes - 1) // lanes

    def body(in_ref, out_ref):
        inT = in_ref[...].T.reshape(lanes // 4, 4, D)
        for i in range(4):
            out_ref[:, D * i : D * (i + 1)] = inT[:, i, :]

    return pl.pallas_call(
        body,
        grid=(grid,),
        in_specs=[pl.BlockSpec((D, lanes), lambda g: (0, g))],
        out_specs=pl.BlockSpec((lanes * D // 128, 128), lambda g: (g, 0)),
        out_shape=jax.ShapeDtypeStruct((rows_out, 128), jnp.float32),
    )


@functools.lru_cache(maxsize=None)
def _build_gather(n_fields: int, batch: int, vocab: int, nw: int):
    tcols = batch // CHUNK
    n_chunks = n_fields * tcols
    per_w = n_chunks // nw
    assert n_chunks % nw == 0 and LOOKAHEAD < NBUF
    assert (per_w - 2 * LOOKAHEAD) % NBUF == 0 and NBUF % OB == 0
    mesh = plsc.VectorSubcoreMesh(core_axis_name="c", subcore_axis_name="s")

    @functools.partial(
        pl.kernel,
        mesh=mesh,
        out_type=jax.ShapeDtypeStruct(
            (n_fields, D // 8, tcols, 8, CHUNK), jnp.float32
        ),
        scratch_types=[
            pltpu.VMEM((per_w, CHUNK), jnp.int32),
            *[pltpu.VMEM((CHUNK, D), jnp.float32) for _ in range(NBUF)],
            *[pltpu.VMEM((D, CHUNK), jnp.float32) for _ in range(OB)],
            *[pltpu.SemaphoreType.DMA for _ in range(NBUF + OB)],
        ],
        compiler_params=pltpu.CompilerParams(
            use_tc_tiling_on_sc=False, needs_layout_passes=False
        ),
    )
    def embed_kernel(idx_hbm, table_hbm, out_hbm, idx_v, *rest):
        rows = rest[:NBUF]
        obufs = rest[NBUF : NBUF + OB]
        g_sems = rest[NBUF + OB : 2 * NBUF + OB]
        w_sems = rest[2 * NBUF + OB : 2 * NBUF + 2 * OB]
        wid = lax.axis_index("s") * 2 + lax.axis_index("c")
        base = wid * per_w

        pltpu.sync_copy(idx_hbm.at[pl.ds(base, per_w)], idx_v)

        iota = lax.iota(jnp.int32, 16)

        def gstart(j, b):
            pltpu.async_copy(table_hbm.at[idx_v.at[j]], rows[b], g_sems[b])

        def gwait(j, b):
            pltpu.make_async_copy(
                table_hbm.at[idx_v.at[j]], rows[b], g_sems[b]
            ).wait()

        def transpose(b, ob):
            src = rows[b]
            dst = obufs[ob]

            def tbody(ci, carry):
                for cc in range(4):
                    c = ci * 4 + cc
                    cvec = jnp.full((16,), c, jnp.int32)
                    for k in range(CHUNK // 16):
                        bvec = iota + (16 * k)
                        dst[c, 16 * k : 16 * (k + 1)] = plsc.load_gather(
                            src, [bvec, cvec]
                        )
                return carry

            lax.fori_loop(0, D // 4, tbody, 0)

        def out_tiles(j):
            g_id = base + j
            f = g_id // tcols
            tcol = g_id % tcols
            return [out_hbm.at[f, tr, tcol] for tr in range(D // 8)]

        def wstart(j, ob):
            for tr, dstt in enumerate(out_tiles(j)):
                pltpu.async_copy(
                    obufs[ob].at[pl.ds(tr * 8, 8)], dstt, w_sems[ob]
                )

        def wdrain(j, ob):
            for tr, dstt in enumerate(out_tiles(j)):
                pltpu.make_async_copy(
                    obufs[ob].at[pl.ds(tr * 8, 8)], dstt, w_sems[ob]
                ).wait()

        def step(j, b, ob, drain, launch_b):
            gwait(j, b)
            if drain:
                wdrain(j - OB, ob)
            transpose(b, ob)
            wstart(j, ob)
            if launch_b is not None:
                gstart(j + LOOKAHEAD, launch_b)

        for j in range(LOOKAHEAD):
            gstart(j, j % NBUF)
        for j in range(LOOKAHEAD):
            step(
                j, j % NBUF, j % OB,
                drain=j >= OB,
                launch_b=(j + LOOKAHEAD) % NBUF,
            )

        def body(g, carry):
            for u in range(NBUF):
                j = g * NBUF + LOOKAHEAD + u
                step(
                    j,
                    (LOOKAHEAD + u) % NBUF,
                    (LOOKAHEAD + u) % OB,
                    drain=True,
                    launch_b=(2 * LOOKAHEAD + u) % NBUF,
                )
            return carry

        lax.fori_loop(0, (per_w - 2 * LOOKAHEAD) // NBUF, body, 0)

        for j in range(per_w - LOOKAHEAD, per_w):
            step(j, j % NBUF, j % OB, drain=True, launch_b=None)
        for j in range(per_w - OB, per_w):
            wdrain(j, j % OB)

    return embed_kernel


def kernel(x, table):
    batch, n_fields = x.shape
    vocab = table.shape[0]
    info = plsc.get_sparse_core_info()
    nw = info.num_cores * info.num_subcores
    idx = x.T.astype(jnp.int32).reshape(n_fields * (batch // CHUNK), CHUNK)
    tbl_lin = _build_table_detranspose(vocab)(table.T).reshape(vocab, D)
    k5 = _build_gather(n_fields, batch, vocab, nw)(idx, tbl_lin)
    return k5.transpose(2, 4, 0, 1, 3).reshape(batch, n_fields, D)


# speedup vs baseline: 1.0668x; 1.0668x over previous
import functools

import jax
import jax.numpy as jnp
from jax import lax
from jax.experimental import pallas as pl
from jax.experimental.pallas import tpu as pltpu
from jax.experimental.pallas import tpu_sc as plsc

D = 32
CHUNK = 128
LOOKAHEAD = 4
NBUF = 6
OB = 2


@functools.lru_cache(maxsize=None)
def _build_table_detranspose(vocab: int):
    lanes = 1024
    rows_out = vocab * D // 128
    grid = (vocab + lanes - 1) // lanes

    def body(in_ref, out_ref):
        inT = in_ref[...].T.reshape(lanes // 4, 4, D)
        for i in range(4):
            out_ref[:, D * i : D * (i + 1)] = inT[:, i, :]

    return pl.pallas_call(
        body,
        grid=(grid,),
        in_specs=[pl.BlockSpec((D, lanes), lambda g: (0, g))],
        out_specs=pl.BlockSpec((lanes * D // 128, 128), lambda g: (g, 0)),
        out_shape=jax.ShapeDtypeStruct((rows_out, 128), jnp.float32),
    )


@functools.lru_cache(maxsize=None)
def _build_gather(n_fields: int, batch: int, vocab: int, nw: int):
    tcols = batch // CHUNK
    n_chunks = n_fields * tcols
    per_w = n_chunks // nw
    assert n_chunks % nw == 0 and LOOKAHEAD < NBUF
    assert (per_w - 2 * LOOKAHEAD) % NBUF == 0 and NBUF % OB == 0
    mesh = plsc.VectorSubcoreMesh(core_axis_name="c", subcore_axis_name="s")

    @functools.partial(
        pl.kernel,
        mesh=mesh,
        out_type=jax.ShapeDtypeStruct(
            (n_fields, D // 8, tcols, 8 * CHUNK), jnp.float32
        ),
        scratch_types=[
            pltpu.VMEM((per_w, CHUNK), jnp.int32),
            *[pltpu.VMEM((CHUNK, D), jnp.float32) for _ in range(NBUF)],
            *[pltpu.VMEM((D * CHUNK,), jnp.float32) for _ in range(OB)],
            *[pltpu.SemaphoreType.DMA for _ in range(NBUF + OB)],
        ],
        compiler_params=pltpu.CompilerParams(
            use_tc_tiling_on_sc=False, needs_layout_passes=False
        ),
    )
    def embed_kernel(idx_hbm, table_hbm, out_hbm, idx_v, *rest):
        rows = rest[:NBUF]
        obufs = rest[NBUF : NBUF + OB]
        g_sems = rest[NBUF + OB : 2 * NBUF + OB]
        w_sems = rest[2 * NBUF + OB : 2 * NBUF + 2 * OB]
        wid = lax.axis_index("s") * 2 + lax.axis_index("c")
        base = wid * per_w

        pltpu.sync_copy(idx_hbm.at[pl.ds(base, per_w)], idx_v)

        iota = lax.iota(jnp.int32, 16)

        def gstart(j, b):
            pltpu.async_copy(table_hbm.at[idx_v.at[j]], rows[b], g_sems[b])

        def gwait(j, b):
            pltpu.make_async_copy(
                table_hbm.at[idx_v.at[j]], rows[b], g_sems[b]
            ).wait()

        cvecs128 = [(iota + (16 * h)) * CHUNK for h in range(D // 16)]

        def transpose(b, ob):
            src = rows[b]
            dst = obufs[ob]

            def tbody(li, carry):
                for lu in range(4):
                    l = li * 4 + lu
                    for h in range(D // 16):
                        plsc.store_scatter(
                            dst, [cvecs128[h] + l],
                            src[l, 16 * h : 16 * (h + 1)],
                        )
                return carry

            lax.fori_loop(0, CHUNK // 4, tbody, 0)

        def out_tiles(j):
            g_id = base + j
            f = g_id // tcols
            tcol = g_id % tcols
            return [out_hbm.at[f, tr, tcol] for tr in range(D // 8)]

        def wstart(j, ob):
            for tr, dstt in enumerate(out_tiles(j)):
                pltpu.async_copy(
                    obufs[ob].at[pl.ds(tr * 8 * CHUNK, 8 * CHUNK)],
                    dstt, w_sems[ob],
                )

        def wdrain(j, ob):
            for tr, dstt in enumerate(out_tiles(j)):
                pltpu.make_async_copy(
                    obufs[ob].at[pl.ds(tr * 8 * CHUNK, 8 * CHUNK)],
                    dstt, w_sems[ob],
                ).wait()

        def step(j, b, ob, drain, launch_b):
            gwait(j, b)
            if drain:
                wdrain(j - OB, ob)
            transpose(b, ob)
            wstart(j, ob)
            if launch_b is not None:
                gstart(j + LOOKAHEAD, launch_b)

        for j in range(LOOKAHEAD):
            gstart(j, j % NBUF)
        for j in range(LOOKAHEAD):
            step(
                j, j % NBUF, j % OB,
                drain=j >= OB,
                launch_b=(j + LOOKAHEAD) % NBUF,
            )

        def body(g, carry):
            for u in range(NBUF):
                j = g * NBUF + LOOKAHEAD + u
                step(
                    j,
                    (LOOKAHEAD + u) % NBUF,
                    (LOOKAHEAD + u) % OB,
                    drain=True,
                    launch_b=(2 * LOOKAHEAD + u) % NBUF,
                )
            return carry

        lax.fori_loop(0, (per_w - 2 * LOOKAHEAD) // NBUF, body, 0)

        for j in range(per_w - LOOKAHEAD, per_w):
            step(j, j % NBUF, j % OB, drain=True, launch_b=None)
        for j in range(per_w - OB, per_w):
            wdrain(j, j % OB)

    return embed_kernel


def kernel(x, table):
    batch, n_fields = x.shape
    vocab = table.shape[0]
    info = plsc.get_sparse_core_info()
    nw = info.num_cores * info.num_subcores
    idx = x.T.astype(jnp.int32).reshape(n_fields * (batch // CHUNK), CHUNK)
    tbl_lin = _build_table_detranspose(vocab)(table.T).reshape(vocab, D)
    k4 = _build_gather(n_fields, batch, vocab, nw)(idx, tbl_lin)
    k5 = k4.reshape(n_fields, D // 8, batch // CHUNK, 8, CHUNK)
    return k5.transpose(2, 4, 0, 1, 3).reshape(batch, n_fields, D)


# speedup vs baseline: 1.3730x; 1.2871x over previous
import functools

import jax
import jax.numpy as jnp
from jax import lax
from jax.experimental import pallas as pl
from jax.experimental.pallas import tpu as pltpu
from jax.experimental.pallas import tpu_sc as plsc

D = 32
CHUNK = 128
LOOKAHEAD = 4
NBUF = 6
OB = 2


@functools.lru_cache(maxsize=None)
def _build_table_detranspose(vocab: int):
    lanes = 1024
    rows_out = vocab * D // 128
    grid = (vocab + lanes - 1) // lanes

    def body(in_ref, out_ref):
        inT = in_ref[...].T.reshape(lanes // 4, 4, D)
        for i in range(4):
            out_ref[:, D * i : D * (i + 1)] = inT[:, i, :]

    return pl.pallas_call(
        body,
        grid=(grid,),
        in_specs=[pl.BlockSpec((D, lanes), lambda g: (0, g))],
        out_specs=pl.BlockSpec((lanes * D // 128, 128), lambda g: (g, 0)),
        out_shape=jax.ShapeDtypeStruct((rows_out, 128), jnp.float32),
    )


@functools.lru_cache(maxsize=None)
def _build_gather(n_fields: int, batch: int, vocab: int, nw: int):
    tcols = batch // CHUNK
    n_chunks = n_fields * tcols
    per_w = n_chunks // nw
    assert n_chunks % nw == 0 and LOOKAHEAD < NBUF
    assert (per_w - 2 * LOOKAHEAD) % NBUF == 0 and NBUF % OB == 0
    mesh = plsc.VectorSubcoreMesh(core_axis_name="c", subcore_axis_name="s")

    @functools.partial(
        pl.kernel,
        mesh=mesh,
        out_type=jax.ShapeDtypeStruct(
            (n_fields, D // 8, tcols, 8 * CHUNK), jnp.float32
        ),
        scratch_types=[
            pltpu.VMEM((per_w, CHUNK), jnp.int32),
            *[pltpu.VMEM((CHUNK, D), jnp.float32) for _ in range(NBUF)],
            *[pltpu.VMEM((D * CHUNK,), jnp.float32) for _ in range(OB)],
            *[pltpu.SemaphoreType.DMA for _ in range(NBUF + OB)],
        ],
        compiler_params=pltpu.CompilerParams(
            use_tc_tiling_on_sc=False, needs_layout_passes=False
        ),
    )
    def embed_kernel(idx_hbm, table_hbm, out_hbm, idx_v, *rest):
        rows = rest[:NBUF]
        obufs = rest[NBUF : NBUF + OB]
        g_sems = rest[NBUF + OB : 2 * NBUF + OB]
        w_sems = rest[2 * NBUF + OB : 2 * NBUF + 2 * OB]
        wid = lax.axis_index("s") * 2 + lax.axis_index("c")
        base = wid * per_w

        pltpu.sync_copy(idx_hbm.at[pl.ds(base, per_w)], idx_v)

        iota = lax.iota(jnp.int32, 16)

        def gstart(j, b):
            pltpu.async_copy(table_hbm.at[idx_v.at[j]], rows[b], g_sems[b])

        def gwait(j, b):
            pltpu.make_async_copy(
                table_hbm.at[idx_v.at[j]], rows[b], g_sems[b]
            ).wait()

        cvecs128 = [(iota + (16 * h)) * CHUNK for h in range(D // 16)]

        def transpose(b, ob):
            src = rows[b]
            dst = obufs[ob]

            def tbody(li, carry):
                for lu in range(4):
                    l = li * 4 + lu
                    for h in range(D // 16):
                        plsc.store_scatter(
                            dst, [cvecs128[h] + l],
                            src[l, 16 * h : 16 * (h + 1)],
                        )
                return carry

            lax.fori_loop(0, CHUNK // 4, tbody, 0)

        def out_tiles(j):
            g_id = base + j
            f = g_id // tcols
            tcol = g_id % tcols
            return [out_hbm.at[f, tr, tcol] for tr in range(D // 8)]

        def wstart(j, ob):
            for tr, dstt in enumerate(out_tiles(j)):
                pltpu.async_copy(
                    obufs[ob].at[pl.ds(tr * 8 * CHUNK, 8 * CHUNK)],
                    dstt, w_sems[ob],
                )

        def wdrain(j, ob):
            for tr, dstt in enumerate(out_tiles(j)):
                pltpu.make_async_copy(
                    obufs[ob].at[pl.ds(tr * 8 * CHUNK, 8 * CHUNK)],
                    dstt, w_sems[ob],
                ).wait()

        def step(j, b, ob, drain, launch_b):
            gwait(j, b)
            if drain:
                wdrain(j - OB, ob)
            transpose(b, ob)
            wstart(j, ob)
            if launch_b is not None:
                gstart(j + LOOKAHEAD, launch_b)

        for j in range(LOOKAHEAD):
            gstart(j, j % NBUF)
        for j in range(LOOKAHEAD):
            step(
                j, j % NBUF, j % OB,
                drain=j >= OB,
                launch_b=(j + LOOKAHEAD) % NBUF,
            )

        def body(g, carry):
            for u in range(NBUF):
                j = g * NBUF + LOOKAHEAD + u
                step(
                    j,
                    (LOOKAHEAD + u) % NBUF,
                    (LOOKAHEAD + u) % OB,
                    drain=True,
                    launch_b=(2 * LOOKAHEAD + u) % NBUF,
                )
            return carry

        lax.fori_loop(0, (per_w - 2 * LOOKAHEAD) // NBUF, body, 0)

        for j in range(per_w - LOOKAHEAD, per_w):
            step(j, j % NBUF, j % OB, drain=True, launch_b=None)
        for j in range(per_w - OB, per_w):
            wdrain(j, j % OB)

    return embed_kernel


def kernel(x, table):
    batch, n_fields = x.shape
    vocab = table.shape[0]
    info = plsc.get_sparse_core_info()
    nw = info.num_cores * info.num_subcores
    idx = x.T.astype(jnp.int32).reshape(n_fields * (batch // CHUNK), CHUNK)
    k4 = _build_gather(n_fields, batch, vocab, nw)(idx, table)
    k5 = k4.reshape(n_fields, D // 8, batch // CHUNK, 8, CHUNK)
    return k5.transpose(2, 4, 0, 1, 3).reshape(batch, n_fields, D)
